# baseline (speedup 1.0000x reference)
import functools

import jax
import jax.numpy as jnp
from jax import lax
from jax.experimental import pallas as pl
from jax.experimental.pallas import tpu as pltpu
from jax.experimental.pallas import tpu_sc as plsc

_NUM_CORES = 2
_NUM_SUBCORES = 16
_NW = _NUM_CORES * _NUM_SUBCORES

_L = 16
_BT = 128
_NBUF = 4


def _make_sc_gather(hist: int, d: int, batch: int):
    assert batch == _NW * _NBUF * _BT
    nd8 = d // 8
    mesh = plsc.VectorSubcoreMesh(core_axis_name="c", subcore_axis_name="s")

    @functools.partial(
        pl.kernel,
        mesh=mesh,
        out_type=jax.ShapeDtypeStruct((hist, nd8, batch // _BT, 8, _BT),
                                      jnp.float32),
        compiler_params=pltpu.CompilerParams(
            use_tc_tiling_on_sc=False, needs_layout_passes=False,
        ),
        scratch_types=(
            [pltpu.VMEM((hist, _NBUF * _BT), jnp.int32)]
            + [pltpu.VMEM((_BT, d), jnp.float32) for _ in range(_NBUF)]
            + [pltpu.VMEM((nd8, 8, _BT), jnp.float32) for _ in range(_NBUF)]
            + [pltpu.SemaphoreType.DMA for _ in range(2 * _NBUF)]
        ),
    )
    def gather_kernel(table_hbm, xt_hbm, out_hbm, idx_v, *rest):
        bufs = rest[:_NBUF]
        tbufs = rest[_NBUF:2 * _NBUF]
        gsem = rest[2 * _NBUF:3 * _NBUF]
        ssem = rest[3 * _NBUF:]
        wid = lax.axis_index("s") * _NUM_CORES + lax.axis_index("c")

        pltpu.sync_copy(xt_hbm.at[:, pl.ds(wid * (_NBUF * _BT), _NBUF * _BT)],
                        idx_v)

        def gather_start(j, b):
            pltpu.make_async_copy(
                table_hbm.at[idx_v.at[j, pl.ds(b * _BT, _BT)]],
                bufs[b], gsem[b],
            ).start()

        def gather_wait(j, b):
            pltpu.make_async_copy(
                table_hbm.at[idx_v.at[j, pl.ds(b * _BT, _BT)]],
                bufs[b], gsem[b],
            ).wait()

        def store_start(j, b):
            pltpu.make_async_copy(
                tbufs[b],
                out_hbm.at[j, :, wid * _NBUF + b], ssem[b],
            ).start()

        def store_wait(j, b):
            pltpu.make_async_copy(
                tbufs[b],
                out_hbm.at[j, :, wid * _NBUF + b], ssem[b],
            ).wait()

        lane = jnp.arange(_L, dtype=jnp.int32)
        dt_idx = [lane // 8 + (c * _L) // 8 for c in range(d // _L)]
        ds_idx = lane % 8

        def transpose_block(b):
            buf, tbuf = bufs[b], tbufs[b]

            @plsc.parallel_loop(0, _BT, unroll=8)
            def _(r):
                rvec = jnp.full((_L,), r, jnp.int32)
                for c in range(d // _L):
                    v = buf[r, pl.ds(c * _L, _L)]
                    plsc.store_scatter(tbuf, [dt_idx[c], ds_idx, rvec], v)

        for b in range(_NBUF):
            gather_start(0, b)

        def j_body(j, carry):
            for b in range(_NBUF):
                gather_wait(j, b)

                @pl.when(j > 0)
                def _():
                    store_wait(j - 1, b)

                transpose_block(b)
                store_start(j, b)

                @pl.when(j < hist - 1)
                def _():
                    gather_start(j + 1, b)

            return carry

        lax.fori_loop(0, hist, j_body, 0)

        for b in range(_NBUF):
            store_wait(hist - 1, b)

    return gather_kernel


def kernel(x, table):
    batch, hist = x.shape
    vocab, d = table.shape
    out5 = _make_sc_gather(hist, d, batch)(table, x.T)
    return out5.transpose(2, 4, 0, 1, 3).reshape(batch, hist, d)


# speedup vs baseline: 1.4751x; 1.4751x over previous
import functools

import jax
import jax.numpy as jnp
from jax import lax
from jax.experimental import pallas as pl
from jax.experimental.pallas import tpu as pltpu
from jax.experimental.pallas import tpu_sc as plsc

_NUM_CORES = 2
_NUM_SUBCORES = 16
_NW = _NUM_CORES * _NUM_SUBCORES

_L = 16
_BT = 128
_NBUF = 4


def _make_sc_gather(hist: int, d: int, batch: int):
    assert batch == _NW * _NBUF * _BT
    nd8 = d // 8
    mesh = plsc.VectorSubcoreMesh(core_axis_name="c", subcore_axis_name="s")

    @functools.partial(
        pl.kernel,
        mesh=mesh,
        out_type=jax.ShapeDtypeStruct((hist, nd8, batch // _BT, 8, _BT),
                                      jnp.float32),
        compiler_params=pltpu.CompilerParams(
            use_tc_tiling_on_sc=False, needs_layout_passes=False,
        ),
        scratch_types=(
            [pltpu.VMEM((hist, _NBUF * _BT), jnp.int32)]
            + [pltpu.VMEM((_BT, d), jnp.float32) for _ in range(_NBUF)]
            + [pltpu.VMEM((nd8, 8, _BT), jnp.float32) for _ in range(_NBUF)]
            + [pltpu.SemaphoreType.DMA for _ in range(2 * _NBUF)]
        ),
    )
    def gather_kernel(table_hbm, xt_hbm, out_hbm, idx_v, *rest):
        bufs = rest[:_NBUF]
        tbufs = rest[_NBUF:2 * _NBUF]
        gsem = rest[2 * _NBUF:3 * _NBUF]
        ssem = rest[3 * _NBUF:]
        wid = lax.axis_index("s") * _NUM_CORES + lax.axis_index("c")

        pltpu.sync_copy(xt_hbm.at[:, pl.ds(wid * (_NBUF * _BT), _NBUF * _BT)],
                        idx_v)

        def gather_start(j, b):
            pltpu.make_async_copy(
                table_hbm.at[idx_v.at[j, pl.ds(b * _BT, _BT)]],
                bufs[b], gsem[b],
            ).start()

        def gather_wait(j, b):
            pltpu.make_async_copy(
                table_hbm.at[idx_v.at[j, pl.ds(b * _BT, _BT)]],
                bufs[b], gsem[b],
            ).wait()

        def store_start(j, b):
            pltpu.make_async_copy(
                tbufs[b],
                out_hbm.at[j, :, wid * _NBUF + b], ssem[b],
            ).start()

        def store_wait(j, b):
            pltpu.make_async_copy(
                tbufs[b],
                out_hbm.at[j, :, wid * _NBUF + b], ssem[b],
            ).wait()

        lane = jnp.arange(_L, dtype=jnp.int32)
        col_idx = [lane + c * _L for c in range(d // _L)]
        dt_idx = [lane // 8 + (c * _L) // 8 for c in range(d // _L)]
        ds_idx = lane % 8

        def transpose_block(b):
            buf, tbuf = bufs[b], tbufs[b]

            @plsc.parallel_loop(0, _BT, unroll=8)
            def _(r0):
                rr = (jnp.full((_L,), r0, jnp.int32) + lane) & (_BT - 1)
                for c in range(d // _L):
                    v = plsc.load_gather(buf, [rr, col_idx[c]])
                    plsc.store_scatter(tbuf, [dt_idx[c], ds_idx, rr], v)

        for b in range(_NBUF):
            gather_start(0, b)

        def j_body(j, carry):
            for b in range(_NBUF):
                gather_wait(j, b)

                @pl.when(j > 0)
                def _():
                    store_wait(j - 1, b)

                transpose_block(b)
                store_start(j, b)

                @pl.when(j < hist - 1)
                def _():
                    gather_start(j + 1, b)

            return carry

        lax.fori_loop(0, hist, j_body, 0)

        for b in range(_NBUF):
            store_wait(hist - 1, b)

    return gather_kernel


def kernel(x, table):
    batch, hist = x.shape
    vocab, d = table.shape
    out5 = _make_sc_gather(hist, d, batch)(table, x.T)
    return out5.transpose(2, 4, 0, 1, 3).reshape(batch, hist, d)


# speedup vs baseline: 3.4011x; 2.3056x over previous
import functools

import jax
import jax.numpy as jnp
from jax import lax
from jax.experimental import pallas as pl
from jax.experimental.pallas import tpu as pltpu
from jax.experimental.pallas import tpu_sc as plsc

_NUM_CORES = 2
_NUM_SUBCORES = 16
_NW = _NUM_CORES * _NUM_SUBCORES

_L = 16
_BT = 128
_NBUF = 4


_VBLK = 128


def _make_sc_convert(vocab: int, d: int):
    n_full = vocab // _VBLK
    mesh = plsc.VectorSubcoreMesh(core_axis_name="c", subcore_axis_name="s")
    kmax = (n_full + _NW - 1) // _NW
    ngrp = (kmax + 1) // 2

    @functools.partial(
        pl.kernel,
        mesh=mesh,
        out_type=jax.ShapeDtypeStruct((vocab * d,), jnp.float32),
        compiler_params=pltpu.CompilerParams(
            use_tc_tiling_on_sc=True, needs_layout_passes=False,
        ),
        scratch_types=(
            [pltpu.VMEM((d, _VBLK), jnp.float32) for _ in range(2)]
            + [pltpu.VMEM((_VBLK * d,), jnp.float32) for _ in range(2)]
            + [pltpu.VMEM((d, vocab - (vocab // _VBLK) * _VBLK or _VBLK), jnp.float32)]
            + [pltpu.SemaphoreType.DMA for _ in range(4)]
        ),
    )
    def convert_kernel(tt_hbm, rm_hbm, in0, in1, out0, out1, tail_in, *sems):
        ins = (in0, in1)
        outs = (out0, out1)
        gsem = sems[:2]
        ssem = sems[2:]
        wid = lax.axis_index("s") * _NUM_CORES + lax.axis_index("c")

        def ct_of(k):
            return wid + k * _NW

        def in_start(k, b):
            pltpu.make_async_copy(
                tt_hbm.at[:, pl.ds(ct_of(k) * _VBLK, _VBLK)], ins[b], gsem[b],
            ).start()

        def in_wait(k, b):
            pltpu.make_async_copy(
                tt_hbm.at[:, pl.ds(ct_of(k) * _VBLK, _VBLK)], ins[b], gsem[b],
            ).wait()

        def out_start(k, b):
            pltpu.make_async_copy(
                outs[b], rm_hbm.at[pl.ds(ct_of(k) * _VBLK * d, _VBLK * d)],
                ssem[b],
            ).start()

        def out_wait(k, b):
            pltpu.make_async_copy(
                outs[b], rm_hbm.at[pl.ds(ct_of(k) * _VBLK * d, _VBLK * d)],
                ssem[b],
            ).wait()

        lane = jnp.arange(_L, dtype=jnp.int32)

        def permute(it, ot, width):
            @plsc.parallel_loop(0, d, unroll=4)
            def _(dlt):
                rr = (lane + dlt) & (d - 1)
                for c8 in range(width // _L):
                    cc = lane + c8 * _L
                    v = plsc.load_gather(it, [rr, cc])
                    plsc.store_scatter(ot, [cc * d + rr], v)

        def permute_block(b):
            permute(ins[b], outs[b], _VBLK)

        for b in range(2):
            @pl.when(ct_of(b) < n_full)
            def _():
                in_start(b, b)

        def g_body(g, carry):
            for b in range(2):
                k = 2 * g + b

                @pl.when(ct_of(k) < n_full)
                def _():
                    in_wait(k, b)

                    @pl.when(k >= 2)
                    def _():
                        out_wait(k - 2, b)

                    permute_block(b)
                    out_start(k, b)

                    @pl.when(ct_of(k + 2) < n_full)
                    def _():
                        in_start(k + 2, b)

            return carry

        lax.fori_loop(0, ngrp, g_body, 0)

        for b in range(2):
            @pl.when(ct_of(b) < n_full)
            def _():
                out_wait(b, b)

        tail_w = vocab - n_full * _VBLK
        if tail_w:
            @pl.when(wid == _NW - 1)
            def _():
                pltpu.sync_copy(
                    tt_hbm.at[:, pl.ds(n_full * _VBLK, tail_w)], tail_in)
                permute(tail_in, outs[0], tail_w)
                pltpu.sync_copy(
                    outs[0].at[pl.ds(0, tail_w * d)],
                    rm_hbm.at[pl.ds(n_full * _VBLK * d, tail_w * d)])

    return convert_kernel


def _make_sc_gather(hist: int, d: int, batch: int):
    assert batch == _NW * _NBUF * _BT
    nd8 = d // 8
    mesh = plsc.VectorSubcoreMesh(core_axis_name="c", subcore_axis_name="s")

    @functools.partial(
        pl.kernel,
        mesh=mesh,
        out_type=jax.ShapeDtypeStruct((hist, nd8, batch // _BT, 8, _BT),
                                      jnp.float32),
        compiler_params=pltpu.CompilerParams(
            use_tc_tiling_on_sc=False, needs_layout_passes=False,
        ),
        scratch_types=(
            [pltpu.VMEM((hist, _NBUF * _BT), jnp.int32)]
            + [pltpu.VMEM((_BT, d), jnp.float32) for _ in range(_NBUF)]
            + [pltpu.VMEM((nd8, 8, _BT), jnp.float32) for _ in range(_NBUF)]
            + [pltpu.SemaphoreType.DMA for _ in range(2 * _NBUF)]
        ),
    )
    def gather_kernel(table_hbm, xt_hbm, out_hbm, idx_v, *rest):
        bufs = rest[:_NBUF]
        tbufs = rest[_NBUF:2 * _NBUF]
        gsem = rest[2 * _NBUF:3 * _NBUF]
        ssem = rest[3 * _NBUF:]
        wid = lax.axis_index("s") * _NUM_CORES + lax.axis_index("c")

        pltpu.sync_copy(xt_hbm.at[:, pl.ds(wid * (_NBUF * _BT), _NBUF * _BT)],
                        idx_v)

        def gather_start(j, b):
            pltpu.make_async_copy(
                table_hbm.at[idx_v.at[j, pl.ds(b * _BT, _BT)]],
                bufs[b], gsem[b],
            ).start()

        def gather_wait(j, b):
            pltpu.make_async_copy(
                table_hbm.at[idx_v.at[j, pl.ds(b * _BT, _BT)]],
                bufs[b], gsem[b],
            ).wait()

        def store_start(j, b):
            pltpu.make_async_copy(
                tbufs[b],
                out_hbm.at[j, :, wid * _NBUF + b], ssem[b],
            ).start()

        def store_wait(j, b):
            pltpu.make_async_copy(
                tbufs[b],
                out_hbm.at[j, :, wid * _NBUF + b], ssem[b],
            ).wait()

        lane = jnp.arange(_L, dtype=jnp.int32)
        col_idx = [lane + c * _L for c in range(d // _L)]
        dt_idx = [lane // 8 + (c * _L) // 8 for c in range(d // _L)]
        ds_idx = lane % 8

        def transpose_block(b):
            buf, tbuf = bufs[b], tbufs[b]

            @plsc.parallel_loop(0, _BT, unroll=8)
            def _(r0):
                rr = (jnp.full((_L,), r0, jnp.int32) + lane) & (_BT - 1)
                for c in range(d // _L):
                    v = plsc.load_gather(buf, [rr, col_idx[c]])
                    plsc.store_scatter(tbuf, [dt_idx[c], ds_idx, rr], v)

        for b in range(_NBUF):
            gather_start(0, b)

        def j_body(j, carry):
            for b in range(_NBUF):
                gather_wait(j, b)

                @pl.when(j > 0)
                def _():
                    store_wait(j - 1, b)

                transpose_block(b)
                store_start(j, b)

                @pl.when(j < hist - 1)
                def _():
                    gather_start(j + 1, b)

            return carry

        lax.fori_loop(0, hist, j_body, 0)

        for b in range(_NBUF):
            store_wait(hist - 1, b)

    return gather_kernel


def kernel(x, table):
    batch, hist = x.shape
    vocab, d = table.shape
    rm = _make_sc_convert(vocab, d)(table.T)
    out5 = _make_sc_gather(hist, d, batch)(rm.reshape(vocab, d), x.T)
    return out5.transpose(2, 4, 0, 1, 3).reshape(batch, hist, d)
